# baseline (speedup 1.0000x reference)
import functools
import math

import jax
import jax.numpy as jnp
from jax import lax
from jax.experimental import pallas as pl
from jax.experimental.pallas import tpu as pltpu
from jax.experimental.pallas import tpu_sc as plsc

D_MODEL = 128
LANES = 16
CHUNK = 64
NBUF = 4
NSH = 2
NUM_CORES = 2
NUM_SUBCORES = 16
NUM_WORKERS = NUM_CORES * NUM_SUBCORES


def _body(x_hbm, idx_hbm, msk_hbm, pe_hbm, out_hbm,
          pe_sh, x_sh, idx_v, msk_v, eff_v, x_v, rows_v,
          sem_i, sem_h, sem_g, sem_x, sem_o):
    n_tokens = idx_hbm.shape[0]
    per_w = n_tokens // NUM_WORKERS
    n_chunks = per_w // CHUNK
    scale = math.sqrt(float(D_MODEL))
    pad = pe_hbm.shape[0] - 1

    sid = lax.axis_index("s")
    wid = sid * NUM_CORES + lax.axis_index("c")
    base_w = wid * per_w

    @pl.when(sid == 0)
    def _():
        pltpu.sync_copy(pe_hbm, pe_sh)

    plsc.subcore_barrier()

    def issue_idx(c, b):
        base = base_w + c * CHUNK
        pltpu.async_copy(idx_hbm.at[pl.ds(base, CHUNK)], idx_v.at[b], sem_i.at[b])
        pltpu.async_copy(msk_hbm.at[pl.ds(base, CHUNK)], msk_v.at[b], sem_i.at[b])

    def wait_idx(c, b):
        base = base_w + c * CHUNK
        pltpu.make_async_copy(
            idx_hbm.at[pl.ds(base, CHUNK)], idx_v.at[b], sem_i.at[b]).wait()
        pltpu.make_async_copy(
            msk_hbm.at[pl.ds(base, CHUNK)], msk_v.at[b], sem_i.at[b]).wait()

    def issue_hbm2sp(c, p):
        gc = wid * n_chunks + c
        pltpu.async_copy(x_hbm.at[gc], x_sh.at[sid, p], sem_h.at[p])

    def wait_hbm2sp(c, p):
        gc = wid * n_chunks + c
        pltpu.make_async_copy(x_hbm.at[gc], x_sh.at[sid, p], sem_h.at[p]).wait()

    def issue_in(c, b, p):
        pltpu.async_copy(pe_sh.at[eff_v.at[b]], rows_v.at[b], sem_g.at[b])
        pltpu.async_copy(x_sh.at[sid, p], x_v.at[b], sem_x.at[b])

    def wait_in(c, b, p):
        pltpu.make_async_copy(
            pe_sh.at[eff_v.at[b]], rows_v.at[b], sem_g.at[b]).wait()
        pltpu.make_async_copy(x_sh.at[sid, p], x_v.at[b], sem_x.at[b]).wait()

    def issue_out(c, b):
        gc = wid * n_chunks + c
        pltpu.async_copy(rows_v.at[b], out_hbm.at[gc], sem_o.at[b])

    def wait_out(c, b):
        gc = wid * n_chunks + c
        pltpu.make_async_copy(rows_v.at[b], out_hbm.at[gc], sem_o.at[b]).wait()

    def compute_eff(b):
        @plsc.parallel_loop(0, CHUNK, step=LANES)
        def _eff(i):
            sl = pl.ds(i, LANES)
            m = msk_v[b, sl]
            eff_v[b, sl] = jnp.where(m != 0, pad, jnp.minimum(idx_v[b, sl], pad))

    def stage_b(c):
        b = c % NBUF
        p = c % NSH

        wait_idx(c, b)
        compute_eff(b)

        @pl.when(c >= NBUF)
        def _():
            wait_out(c - NBUF, b)

        wait_hbm2sp(c, p)
        issue_in(c, b, p)

    issue_idx(0, 0)
    issue_idx(1, 1)
    issue_idx(2, 2)
    issue_hbm2sp(0, 0)
    issue_hbm2sp(1, 1)
    stage_b(0)

    def outer(g, carry):
        for b in range(NBUF):
            c = NBUF * g + b
            s3 = (b + 3) % NBUF

            @pl.when(c + 3 < n_chunks)
            def _():
                issue_idx(c + 3, s3)

            wait_in(c, b, b % NSH)

            @pl.when(c + 2 < n_chunks)
            def _():
                issue_hbm2sp(c + 2, b % NSH)

            @pl.when(c + 1 < n_chunks)
            def _():
                stage_b(c + 1)

            @plsc.parallel_loop(0, CHUNK, unroll=4)
            def _fma(t):
                for j in range(D_MODEL // LANES):
                    sl = pl.ds(j * LANES, LANES)
                    plsc.addupdate(rows_v.at[b, t, sl], x_v[b, t, sl] * scale)

            issue_out(c, b)
        return carry

    lax.fori_loop(0, n_chunks // NBUF, outer, 0)
    for k in range(min(NBUF, n_chunks), 0, -1):
        wait_out(n_chunks - k, (n_chunks - k) % NBUF)


def kernel(x, mask, indices, pe):
    b, s, d = x.shape
    n = b * s
    x3 = x.reshape(n // CHUNK, CHUNK, d)
    idx = indices.reshape(n).astype(jnp.int32)
    msk = mask.reshape(n).astype(jnp.int32)
    pe_eff = pe.at[pe.shape[0] - 1].set(0.0)

    mesh = plsc.VectorSubcoreMesh(core_axis_name="c", subcore_axis_name="s")
    run = functools.partial(
        pl.kernel,
        mesh=mesh,
        out_type=jax.ShapeDtypeStruct((n // CHUNK, CHUNK, d), jnp.float32),
        scratch_types=[
            pltpu.VMEM_SHARED(pe.shape, jnp.float32),
            pltpu.VMEM_SHARED((NUM_SUBCORES, NSH, CHUNK, D_MODEL), jnp.float32),
            pltpu.VMEM((NBUF, CHUNK), jnp.int32),
            pltpu.VMEM((NBUF, CHUNK), jnp.int32),
            pltpu.VMEM((NBUF, CHUNK), jnp.int32),
            pltpu.VMEM((NBUF, CHUNK, D_MODEL), jnp.float32),
            pltpu.VMEM((NBUF, CHUNK, D_MODEL), jnp.float32),
            pltpu.SemaphoreType.DMA((NBUF,)),
            pltpu.SemaphoreType.DMA((NSH,)),
            pltpu.SemaphoreType.DMA((NBUF,)),
            pltpu.SemaphoreType.DMA((NBUF,)),
            pltpu.SemaphoreType.DMA((NBUF,)),
        ],
    )(_body)
    out = run(x3, idx, msk, pe_eff)
    return out.reshape(b, s, d)


# speedup vs baseline: 1.1813x; 1.1813x over previous
import functools
import math

import jax
import jax.numpy as jnp
from jax import lax
from jax.experimental import pallas as pl
from jax.experimental.pallas import tpu as pltpu
from jax.experimental.pallas import tpu_sc as plsc

D_MODEL = 128
LANES = 16
CHUNK = 80
NBUF = 4
NUM_CORES = 2
NUM_SUBCORES = 16
NUM_WORKERS = NUM_CORES * NUM_SUBCORES


def _body(x_hbm, idx_hbm, msk_hbm, pe_hbm, out_hbm,
          pe_sh, idx_v, msk_v, eff_v, x_v, rows_v,
          sem_i, sem_g, sem_x, sem_o):
    n_tokens = idx_hbm.shape[0]
    per_w = n_tokens // NUM_WORKERS
    n_chunks = per_w // CHUNK
    scale = math.sqrt(float(D_MODEL))
    pad = pe_hbm.shape[0] - 1

    sid = lax.axis_index("s")
    wid = sid * NUM_CORES + lax.axis_index("c")
    base_w = wid * per_w

    @pl.when(sid == 0)
    def _():
        pltpu.sync_copy(pe_hbm, pe_sh)

    plsc.subcore_barrier()

    def issue_idx(c, b):
        base = base_w + c * CHUNK
        pltpu.async_copy(idx_hbm.at[pl.ds(base, CHUNK)], idx_v.at[b], sem_i.at[b])
        pltpu.async_copy(msk_hbm.at[pl.ds(base, CHUNK)], msk_v.at[b], sem_i.at[b])

    def wait_idx(c, b):
        base = base_w + c * CHUNK
        pltpu.make_async_copy(
            idx_hbm.at[pl.ds(base, CHUNK)], idx_v.at[b], sem_i.at[b]).wait()
        pltpu.make_async_copy(
            msk_hbm.at[pl.ds(base, CHUNK)], msk_v.at[b], sem_i.at[b]).wait()

    def issue_in(c, b):
        gc = wid * n_chunks + c
        pltpu.async_copy(pe_sh.at[eff_v.at[b]], rows_v.at[b], sem_g.at[b])
        pltpu.async_copy(x_hbm.at[gc], x_v.at[b], sem_x.at[b])

    def wait_in(c, b):
        gc = wid * n_chunks + c
        pltpu.make_async_copy(
            pe_sh.at[eff_v.at[b]], rows_v.at[b], sem_g.at[b]).wait()
        pltpu.make_async_copy(x_hbm.at[gc], x_v.at[b], sem_x.at[b]).wait()

    def issue_out(c, b):
        gc = wid * n_chunks + c
        pltpu.async_copy(rows_v.at[b], out_hbm.at[gc], sem_o.at[b])

    def wait_out(c, b):
        gc = wid * n_chunks + c
        pltpu.make_async_copy(rows_v.at[b], out_hbm.at[gc], sem_o.at[b]).wait()

    def compute_eff(b):
        @plsc.parallel_loop(0, CHUNK, step=LANES)
        def _eff(i):
            sl = pl.ds(i, LANES)
            m = msk_v[b, sl]
            eff_v[b, sl] = jnp.where(m != 0, pad, jnp.minimum(idx_v[b, sl], pad))

    def stage_b(c):
        b = c % NBUF

        wait_idx(c, b)
        compute_eff(b)

        @pl.when(c >= NBUF)
        def _():
            wait_out(c - NBUF, b)

        issue_in(c, b)

    issue_idx(0, 0)
    issue_idx(1, 1)
    issue_idx(2, 2)
    stage_b(0)
    stage_b(1)

    def outer(g, carry):
        for b in range(NBUF):
            c = NBUF * g + b
            s3 = (b + 3) % NBUF

            @pl.when(c + 3 < n_chunks)
            def _():
                issue_idx(c + 3, s3)

            @pl.when(c + 2 < n_chunks)
            def _():
                stage_b(c + 2)

            wait_in(c, b)

            @plsc.parallel_loop(0, CHUNK, unroll=4)
            def _fma(t):
                for j in range(D_MODEL // LANES):
                    sl = pl.ds(j * LANES, LANES)
                    plsc.addupdate(rows_v.at[b, t, sl], x_v[b, t, sl] * scale)

            issue_out(c, b)
        return carry

    lax.fori_loop(0, n_chunks // NBUF, outer, 0)
    for k in range(min(NBUF, n_chunks), 0, -1):
        wait_out(n_chunks - k, (n_chunks - k) % NBUF)


def kernel(x, mask, indices, pe):
    b, s, d = x.shape
    n = b * s
    x3 = x.reshape(n // CHUNK, CHUNK, d)
    idx = indices.reshape(n).astype(jnp.int32)
    msk = mask.reshape(n).astype(jnp.int32)
    pe_eff = pe.at[pe.shape[0] - 1].set(0.0)

    mesh = plsc.VectorSubcoreMesh(core_axis_name="c", subcore_axis_name="s")
    run = functools.partial(
        pl.kernel,
        mesh=mesh,
        out_type=jax.ShapeDtypeStruct((n // CHUNK, CHUNK, d), jnp.float32),
        scratch_types=[
            pltpu.VMEM_SHARED(pe.shape, jnp.float32),
            pltpu.VMEM((NBUF, CHUNK), jnp.int32),
            pltpu.VMEM((NBUF, CHUNK), jnp.int32),
            pltpu.VMEM((NBUF, CHUNK), jnp.int32),
            pltpu.VMEM((NBUF, CHUNK, D_MODEL), jnp.float32),
            pltpu.VMEM((NBUF, CHUNK, D_MODEL), jnp.float32),
            pltpu.SemaphoreType.DMA((NBUF,)),
            pltpu.SemaphoreType.DMA((NBUF,)),
            pltpu.SemaphoreType.DMA((NBUF,)),
            pltpu.SemaphoreType.DMA((NBUF,)),
        ],
    )(_body)
    out = run(x3, idx, msk, pe_eff)
    return out.reshape(b, s, d)
